# baseline (speedup 1.0000x reference)
import functools
import jax
import jax.numpy as jnp
from jax import lax
from jax.experimental import pallas as pl
from jax.experimental.pallas import tpu as pltpu
from jax.experimental.pallas import tpu_sc as plsc

_B = 16384
_L = 100
_D = 64
_NC = 2
_NS = 16
_NW = _NC * _NS
_RPW = _B // _NW
_G = 4
_LANES = 16


_DNUMS = lax.GatherDimensionNumbers(
    offset_dims=(), collapsed_slice_dims=(0,), start_index_map=(0,))


def _bcast_lane(vec, lane):
    idx = jnp.full((_LANES, 1), lane, dtype=jnp.int32)
    return lax.gather(vec, idx, _DNUMS, slice_sizes=(1,),
                      mode=lax.GatherScatterMode.PROMISE_IN_BOUNDS)


def _sc_body(x_hbm, w_hbm, out_hbm, xbuf, wbuf, obuf):
    wid = lax.axis_index("s") * _NC + lax.axis_index("c")
    base = wid * _RPW
    pltpu.sync_copy(w_hbm, wbuf)

    def chunk_body(it, _):
        row0 = base + it * _G
        pltpu.sync_copy(x_hbm.at[pl.ds(row0, _G)], xbuf)

        def row_body(r, _):
            for l0 in (0, 16, 32, 48, 64, 80, 84):
                lo = l0 if l0 != 84 else 96
                xv = xbuf[r, pl.ds(l0, _LANES)]
                xv = jnp.where(jnp.isnan(xv), 0.0, xv)
                for l in range(lo, min(l0 + _LANES, _L)):
                    xs = _bcast_lane(xv, l - l0)
                    for d in range(_D // _LANES):
                        wv = wbuf[pl.ds(l * _D + d * _LANES, _LANES)]
                        obuf[r, l, pl.ds(d * _LANES, _LANES)] = xs * wv
            return _

        lax.fori_loop(0, _G, row_body, 0)
        pltpu.sync_copy(obuf, out_hbm.at[pl.ds(row0, _G)])
        return _

    lax.fori_loop(0, _RPW // _G, chunk_body, 0)


def kernel(x, emb_weight):
    B, L = x.shape
    D = emb_weight.shape[1]
    wflat = emb_weight[1:L + 1].reshape(L * D)
    mesh = plsc.VectorSubcoreMesh(core_axis_name="c", subcore_axis_name="s")
    run = pl.kernel(
        _sc_body,
        mesh=mesh,
        out_type=jax.ShapeDtypeStruct((B, L, D), x.dtype),
        scratch_types=[
            pltpu.VMEM((_G, L), jnp.float32),
            pltpu.VMEM((L * D,), jnp.float32),
            pltpu.VMEM((_G, L, D), jnp.float32),
        ],
    )
    return run(x, wflat)


# speedup vs baseline: 1.0675x; 1.0675x over previous
import jax
import jax.numpy as jnp
from jax import lax
from jax.experimental import pallas as pl
from jax.experimental.pallas import tpu as pltpu
from jax.experimental.pallas import tpu_sc as plsc

_B = 16384
_L = 100
_D = 64
_NC = 2
_NS = 16
_NW = _NC * _NS
_RPW = _B // _NW
_G = 4
_LANES = 16

_DNUMS = lax.GatherDimensionNumbers(
    offset_dims=(), collapsed_slice_dims=(0,), start_index_map=(0,))


def _bcast_lane(vec, lane):
    idx = jnp.full((_LANES, 1), lane, dtype=jnp.int32)
    return lax.gather(vec, idx, _DNUMS, slice_sizes=(1,),
                      mode=lax.GatherScatterMode.PROMISE_IN_BOUNDS)


def _sc_body(x_hbm, w_hbm, out_hbm, xbuf, wbuf, obuf, sem0, sem1):
    wid = lax.axis_index("s") * _NC + lax.axis_index("c")
    base = wid * _RPW
    pltpu.sync_copy(w_hbm, wbuf)
    sems = (sem0, sem1)

    def compute_chunk(it, p):
        row0 = base + it * _G
        pltpu.sync_copy(x_hbm.at[pl.ds(row0, _G)], xbuf.at[p])

        def row_body(r, _):
            for l0 in (0, 16, 32, 48, 64, 80, 84):
                lo = l0 if l0 != 84 else 96
                xv = xbuf[p, r, pl.ds(l0, _LANES)]
                xv = jnp.where(jnp.isnan(xv), 0.0, xv)
                for l in range(lo, min(l0 + _LANES, _L)):
                    xs = _bcast_lane(xv, l - l0)
                    for d in range(_D // _LANES):
                        wv = wbuf[pl.ds(l * _D + d * _LANES, _LANES)]
                        obuf[p, r, l, pl.ds(d * _LANES, _LANES)] = xs * wv
            return _

        lax.fori_loop(0, _G, row_body, 0)
        pltpu.async_copy(obuf.at[p], out_hbm.at[pl.ds(row0, _G)], sems[p])

    def loop_body(i, _):
        for p in (0, 1):
            it = 2 * i + p

            @pl.when(i > 0)
            def _wait():
                pltpu.make_async_copy(
                    obuf.at[p], out_hbm.at[pl.ds(base, _G)], sems[p]).wait()

            compute_chunk(it, p)
        return _

    n_pairs = _RPW // _G // 2
    lax.fori_loop(0, n_pairs, loop_body, 0)
    for p in (0, 1):
        pltpu.make_async_copy(
            obuf.at[p], out_hbm.at[pl.ds(base, _G)], sems[p]).wait()


def kernel(x, emb_weight):
    B, L = x.shape
    D = emb_weight.shape[1]
    wflat = emb_weight[1:L + 1].reshape(L * D)
    mesh = plsc.VectorSubcoreMesh(core_axis_name="c", subcore_axis_name="s")
    run = pl.kernel(
        _sc_body,
        mesh=mesh,
        out_type=jax.ShapeDtypeStruct((B, L, D), x.dtype),
        scratch_types=[
            pltpu.VMEM((2, _G, L), jnp.float32),
            pltpu.VMEM((L * D,), jnp.float32),
            pltpu.VMEM((2, _G, L, D), jnp.float32),
            pltpu.SemaphoreType.DMA,
            pltpu.SemaphoreType.DMA,
        ],
    )
    return run(x, wflat)
